# baseline (speedup 1.0000x reference)
import functools

import jax
import jax.numpy as jnp
from jax import lax
from jax.experimental import pallas as pl
from jax.experimental.pallas import tpu as pltpu
from jax.experimental.pallas import tpu_sc as plsc

N = 10000
E = 320000
NC = 2
NS = 16
NW = NC * NS
CHUNK = 128
C = 80
EPW = C * CHUNK
CAP = NW * EPW
ACC_ROWS = 10240
ZPT = ACC_ROWS // NS
WB_FULL = N // CHUNK

_f32 = jnp.float32


def _zero_rows(buf, d):
    def row(i, _):
        for k in range(d // 16):
            buf[i, pl.ds(k * 16, 16)] = jnp.zeros((16,), _f32)
        return 0
    lax.fori_loop(0, CHUNK, row, 0)


def _ones_rows(buf, d):
    def row(i, _):
        for k in range(d // 16):
            buf[i, pl.ds(k * 16, 16)] = jnp.ones((16,), _f32)
        return 0
    lax.fori_loop(0, CHUNK, row, 0)


def _make_edge_scatter(d, do_gather):
    mesh = plsc.VectorSubcoreMesh(core_axis_name="c", subcore_axis_name="s")
    NB = 8
    scratch = [
        pltpu.VMEM((C, CHUNK), jnp.int32),
        pltpu.VMEM((C, CHUNK), jnp.int32),
        [pltpu.VMEM((CHUNK, d), _f32) for _ in range(NB)],
        pltpu.VMEM_SHARED((ACC_ROWS, d), _f32),
        pltpu.VMEM_SHARED((N, d), _f32) if do_gather else None,
        [pltpu.SemaphoreType.DMA for _ in range(NB)],
        [pltpu.SemaphoreType.DMA for _ in range(NB)],
    ]
    scratch = [s for s in scratch if s is not None]

    def body(u_hbm, src_hbm, dst_hbm, out_hbm, src_v, dst_v, bufs, acc,
             *rest):
        if do_gather:
            ush, gsem, ssem = rest
        else:
            gsem, ssem = rest
        cc = lax.axis_index("c")
        ss = lax.axis_index("s")
        w = cc * NS + ss

        if do_gather:
            pltpu.sync_copy(src_hbm.at[w], src_v)
            for k in range(5):
                g = ss * 5 + k
                r0 = g * CHUNK

                @pl.when(g < WB_FULL)
                def _():
                    pltpu.sync_copy(u_hbm.at[pl.ds(r0, CHUNK)],
                                    ush.at[pl.ds(r0, CHUNK)])

                @pl.when(g == WB_FULL)
                def _():
                    pltpu.sync_copy(u_hbm.at[pl.ds(r0, 16)],
                                    ush.at[pl.ds(r0, 16)])
        pltpu.sync_copy(dst_hbm.at[w], dst_v)

        _zero_rows(bufs[NB - 1], d)
        zbase = ss * ZPT
        for k in range(ZPT // CHUNK):
            pltpu.sync_copy(bufs[NB - 1], acc.at[pl.ds(zbase + k * CHUNK,
                                                       CHUNK)])
        if not do_gather:
            _ones_rows(bufs[0], d)
        plsc.subcore_barrier()

        if do_gather:
            m0 = bufs[0]
            for pj in range(NB // 2):
                pltpu.async_copy(ush.at[src_v.at[pj]], bufs[pj], gsem[pj])

            def step(i, _):
                for b in range(NB):
                    j = NB * i + b
                    kn = (b + NB // 2) % NB

                    @pl.when(j >= NB // 2)
                    def _():
                        pltpu.make_async_copy(
                            bufs[kn], acc.at[dst_v.at[j - NB // 2]],
                            ssem[kn]).wait()

                    @pl.when(j + NB // 2 < C)
                    def _():
                        pltpu.async_copy(ush.at[src_v.at[j + NB // 2]],
                                         bufs[kn], gsem[kn])

                    pltpu.make_async_copy(ush.at[src_v.at[j]], bufs[b],
                                          gsem[b]).wait()
                    pltpu.async_copy(bufs[b], acc.at[dst_v.at[j]], ssem[b],
                                     add=True)
                return 0
            lax.fori_loop(0, C // NB, step, 0)
            for j in range(C - NB // 2, C):
                pltpu.make_async_copy(bufs[j % NB], acc.at[dst_v.at[j]],
                                      ssem[j % NB]).wait()
        else:
            m0 = bufs[0]
            def step(i, _):
                for b in range(8):
                    pltpu.async_copy(m0, acc.at[dst_v.at[8 * i + b]],
                                     ssem[b % NB], add=True)
                for b in range(8):
                    pltpu.make_async_copy(m0, acc.at[dst_v.at[8 * i + b]],
                                          ssem[b % NB]).wait()
                return 0
            lax.fori_loop(0, C // 8, step, 0)

        plsc.subcore_barrier()

        for k in range(5):
            g = ss * 5 + k
            r0 = g * CHUNK

            @pl.when(g < WB_FULL)
            def _():
                pltpu.sync_copy(acc.at[pl.ds(r0, CHUNK)], m0)
                pltpu.sync_copy(m0, out_hbm.at[cc, pl.ds(r0, CHUNK)])

            @pl.when(g == WB_FULL)
            def _():
                pltpu.sync_copy(acc.at[pl.ds(r0, 16)], m0.at[pl.ds(0, 16)])
                pltpu.sync_copy(m0.at[pl.ds(0, 16)],
                                out_hbm.at[cc, pl.ds(r0, 16)])

    return pl.kernel(
        body,
        out_type=jax.ShapeDtypeStruct((NC, N, d), _f32),
        mesh=mesh,
        scratch_types=scratch,
        compiler_params=pltpu.CompilerParams(use_tc_tiling_on_sc=False),
    )


C2 = 160


def _make_edge_scatter_split():
    d = 32
    mesh = plsc.VectorSubcoreMesh(core_axis_name="c", subcore_axis_name="s")
    NB = 8
    scratch = [
        pltpu.VMEM((C2, CHUNK), jnp.int32),
        pltpu.VMEM((C2, CHUNK), jnp.int32),
        [pltpu.VMEM((CHUNK, d), _f32) for _ in range(NB)],
        pltpu.VMEM_SHARED((ACC_ROWS, d), _f32),
        pltpu.VMEM_SHARED((N, d), _f32),
        [pltpu.SemaphoreType.DMA for _ in range(NB)],
        [pltpu.SemaphoreType.DMA for _ in range(NB)],
    ]

    def body(ua_hbm, ub_hbm, src_hbm, dst_hbm, out_hbm, src_v, dst_v, bufs,
             acc, ush, gsem, ssem):
        cc = lax.axis_index("c")
        ss = lax.axis_index("s")

        pltpu.sync_copy(src_hbm.at[ss], src_v)
        for k in range(5):
            g = ss * 5 + k
            r0 = g * CHUNK

            @pl.when(g < WB_FULL)
            def _():
                @pl.when(cc == 0)
                def _():
                    pltpu.sync_copy(ua_hbm.at[pl.ds(r0, CHUNK)],
                                    ush.at[pl.ds(r0, CHUNK)])

                @pl.when(cc == 1)
                def _():
                    pltpu.sync_copy(ub_hbm.at[pl.ds(r0, CHUNK)],
                                    ush.at[pl.ds(r0, CHUNK)])

            @pl.when(g == WB_FULL)
            def _():
                @pl.when(cc == 0)
                def _():
                    pltpu.sync_copy(ua_hbm.at[pl.ds(r0, 16)],
                                    ush.at[pl.ds(r0, 16)])

                @pl.when(cc == 1)
                def _():
                    pltpu.sync_copy(ub_hbm.at[pl.ds(r0, 16)],
                                    ush.at[pl.ds(r0, 16)])
        pltpu.sync_copy(dst_hbm.at[ss], dst_v)

        _zero_rows(bufs[NB - 1], d)
        zbase = ss * ZPT
        for k in range(ZPT // CHUNK):
            pltpu.sync_copy(bufs[NB - 1], acc.at[pl.ds(zbase + k * CHUNK,
                                                       CHUNK)])
        plsc.subcore_barrier()

        m0 = bufs[0]
        for pj in range(NB // 2):
            pltpu.async_copy(ush.at[src_v.at[pj]], bufs[pj], gsem[pj])

        def step(i, _):
            for b in range(NB):
                j = NB * i + b
                kn = (b + NB // 2) % NB

                @pl.when(j >= NB // 2)
                def _():
                    pltpu.make_async_copy(
                        bufs[kn], acc.at[dst_v.at[j - NB // 2]],
                        ssem[kn]).wait()

                @pl.when(j + NB // 2 < C2)
                def _():
                    pltpu.async_copy(ush.at[src_v.at[j + NB // 2]], bufs[kn],
                                     gsem[kn])

                pltpu.make_async_copy(ush.at[src_v.at[j]], bufs[b],
                                      gsem[b]).wait()
                pltpu.async_copy(bufs[b], acc.at[dst_v.at[j]], ssem[b],
                                 add=True)
            return 0
        lax.fori_loop(0, C2 // NB, step, 0)
        for j in range(C2 - NB // 2, C2):
            pltpu.make_async_copy(bufs[j % NB], acc.at[dst_v.at[j]],
                                  ssem[j % NB]).wait()

        plsc.subcore_barrier()

        for k in range(5):
            g = ss * 5 + k
            r0 = g * CHUNK

            @pl.when(g < WB_FULL)
            def _():
                pltpu.sync_copy(acc.at[pl.ds(r0, CHUNK)], m0)
                pltpu.sync_copy(m0, out_hbm.at[cc, pl.ds(r0, CHUNK)])

            @pl.when(g == WB_FULL)
            def _():
                pltpu.sync_copy(acc.at[pl.ds(r0, 16)], m0.at[pl.ds(0, 16)])
                pltpu.sync_copy(m0.at[pl.ds(0, 16)],
                                out_hbm.at[cc, pl.ds(r0, 16)])

    return pl.kernel(
        body,
        out_type=jax.ShapeDtypeStruct((NC, N, 32), _f32),
        mesh=mesh,
        scratch_types=scratch,
        compiler_params=pltpu.CompilerParams(use_tc_tiling_on_sc=False),
    )


_R = 5000


def _dot(a, b):
    return jnp.dot(a, b, preferred_element_type=_f32,
                   precision=lax.Precision.HIGHEST)


def _tc_call(body, ins, out_widths, rows=N, r=_R):
    in_specs = []
    for a, kind in ins:
        if kind == "r":
            in_specs.append(pl.BlockSpec((r, a.shape[1]), lambda i: (i, 0)))
        elif kind == "p":
            in_specs.append(
                pl.BlockSpec((2, r, a.shape[2]), lambda i: (0, i, 0)))
        else:
            in_specs.append(
                pl.BlockSpec(a.shape, lambda i, nd=a.ndim: (0,) * nd))
    out_shape = [jax.ShapeDtypeStruct((rows, wd), _f32) for wd in out_widths]
    out_specs = [pl.BlockSpec((r, wd), lambda i: (i, 0)) for wd in out_widths]
    res = pl.pallas_call(
        body,
        grid=(rows // r,),
        in_specs=in_specs,
        out_specs=out_specs,
        out_shape=out_shape,
    )(*[a for a, _ in ins])
    return res


def _combine(p_ref, dinv, d2, hself):
    p = p_ref[...]
    return dinv * (p[0] + p[1]) + d2 * hself


def _stage0_body(pdeg, x, w1, dinv_o, d2_o, g1_o, u1_o):
    p = pdeg[...]
    deg = p[0, :, 0:1] + p[1, :, 0:1] + 1.0
    dinv = lax.rsqrt(deg)
    dinv_o[...] = dinv
    d2_o[...] = 1.0 / deg
    g1 = _dot(x[...], w1[...])
    g1_o[...] = g1
    u1_o[...] = dinv * g1


def _stageB_body(p, g1, dinv, d2, b1, h_o, u_o):
    h = _combine(p, dinv[...], d2[...], g1[...]) + b1[...]
    h_o[...] = h
    u_o[...] = dinv[...] * h


def _stageCD_body(p, hprev, dinv, d2, w, b, h_o, u_o):
    s = _combine(p, dinv[...], d2[...], hprev[...])
    h = _dot(s, w[...]) + b[...]
    h_o[...] = h
    u_o[...] = dinv[...] * h


def _stageD_body(p, hprev, dinv, d2, w, b, h_o, ua_o, ub_o):
    s = _combine(p, dinv[...], d2[...], hprev[...])
    h = _dot(s, w[...]) + b[...]
    h_o[...] = h
    u = dinv[...] * h
    ua_o[...] = u[:, :32]
    ub_o[...] = u[:, 32:]


def _combine2(p, dinv, d2, hself):
    v = p[...]
    return dinv * jnp.concatenate([v[0], v[1]], axis=1) + d2 * hself


def _stageE_body(p, h3, dinv, d2, w4, b4, w5, g5_o, ua_o, ub_o):
    s = _combine2(p, dinv[...], d2[...], h3[...])
    h4 = jnp.maximum(_dot(s, w4[...]) + b4[...], 0.0)
    g5 = _dot(h4, w5[...])
    g5_o[...] = g5
    u5 = dinv[...] * g5
    ua_o[...] = u5[:, :32]
    ub_o[...] = u5[:, 32:]


def _stageF_body(p, gprev, dinv, d2, bprev, wnext, g_o, u_o):
    h = _combine2(p, dinv[...], d2[...], gprev[...]) + bprev[...]
    g = _dot(h, wnext[...])
    g_o[...] = g
    u_o[...] = dinv[...] * g


def _stageFG_body(p, gprev, dinv, d2, bprev, wnext, g_o, u_o):
    h = _combine(p, dinv[...], d2[...], gprev[...]) + bprev[...]
    g = _dot(h, wnext[...])
    g_o[...] = g
    u_o[...] = dinv[...] * g


def _stageH_body(p, g7, dinv, d2, b7, h_o):
    h_o[...] = _combine(p, dinv[...], d2[...], g7[...]) + b7[...]


def _head_body(z, wl1, bl1, wl2, bl2, y_o):
    t = jnp.maximum(_dot(z[...], wl1[...]) + bl1[...], 0.0)
    y_o[...] = _dot(t, wl2[...]) + bl2[...]


@functools.partial(jax.jit, static_argnums=())
def _run(x, edge_index, W1, b1, W2, b2, W3, b3, W4, b4, W5, b5, W6, b6,
         W7, b7, Wl1, bl1, Wl2, bl2):
    pad = CAP - E
    src = jnp.concatenate(
        [edge_index[0], jnp.zeros((pad,), jnp.int32)]).reshape(NW, C, CHUNK)
    dst = jnp.concatenate(
        [edge_index[1],
         jnp.full((pad,), N, jnp.int32)]).reshape(NW, C, CHUNK)

    scat16 = _make_edge_scatter(16, True)
    scat32 = _make_edge_scatter(32, True)
    scat64s = _make_edge_scatter_split()
    degk = _make_edge_scatter(16, False)
    src16 = src.reshape(NS, C2, CHUNK)
    dst16 = dst.reshape(NS, C2, CHUNK)

    b1r, b2r, b3r, b4r, b5r, b6r, b7r = (
        b.reshape(1, -1) for b in (b1, b2, b3, b4, b5, b6, b7))
    bl1r, bl2r = bl1.reshape(1, -1), bl2.reshape(1, -1)

    dummy16 = jnp.zeros((N, 16), _f32)
    pdeg = degk(dummy16, src, dst)

    dinv, d2, g1, u1 = _tc_call(
        _stage0_body, [(pdeg, "p"), (x, "r"), (W1, "w")], [1, 1, 16, 16])

    p1 = scat16(u1, src, dst)
    h1, u2 = _tc_call(
        _stageB_body,
        [(p1, "p"), (g1, "r"), (dinv, "r"), (d2, "r"), (b1r, "w")], [16, 16])

    p2 = scat16(u2, src, dst)
    h2, u3 = _tc_call(
        _stageCD_body,
        [(p2, "p"), (h1, "r"), (dinv, "r"), (d2, "r"), (W2, "w"), (b2r, "w")],
        [32, 32])

    p3 = scat32(u3, src, dst)
    h3, u4a, u4b = _tc_call(
        _stageD_body,
        [(p3, "p"), (h2, "r"), (dinv, "r"), (d2, "r"), (W3, "w"), (b3r, "w")],
        [64, 32, 32])

    p4 = scat64s(u4a, u4b, src16, dst16)
    g5, u5a, u5b = _tc_call(
        _stageE_body,
        [(p4, "p"), (h3, "r"), (dinv, "r"), (d2, "r"),
         (W4, "w"), (b4r, "w"), (W5, "w")], [64, 32, 32])

    p5 = scat64s(u5a, u5b, src16, dst16)
    g6, u6 = _tc_call(
        _stageF_body,
        [(p5, "p"), (g5, "r"), (dinv, "r"), (d2, "r"),
         (b5r, "w"), (W6, "w")], [32, 32])

    p6 = scat32(u6, src, dst)
    g7, u7 = _tc_call(
        _stageFG_body,
        [(p6, "p"), (g6, "r"), (dinv, "r"), (d2, "r"), (b6r, "w"),
         (W7, "w")], [16, 16])

    p7 = scat16(u7, src, dst)
    (h7,) = _tc_call(
        _stageH_body,
        [(p7, "p"), (g7, "r"), (dinv, "r"), (d2, "r"), (b7r, "w")], [16])

    z = h7.reshape(N // 4, 64)
    (y,) = _tc_call(
        _head_body,
        [(z, "r"), (Wl1, "w"), (bl1r, "w"), (Wl2, "w"), (bl2r, "w")],
        [729], rows=N // 4, r=N // 4)
    return y.reshape(N // 4, 81, 9)


def kernel(x, edge_index, W1, b1, W2, b2, W3, b3, W4, b4, W5, b5, W6, b6,
           W7, b7, Wl1, bl1, Wl2, bl2):
    return _run(x, edge_index, W1, b1, W2, b2, W3, b3, W4, b4, W5, b5,
                W6, b6, W7, b7, Wl1, bl1, Wl2, bl2)


# speedup vs baseline: 1.0275x; 1.0275x over previous
import functools

import jax
import jax.numpy as jnp
from jax import lax
from jax.experimental import pallas as pl
from jax.experimental.pallas import tpu as pltpu
from jax.experimental.pallas import tpu_sc as plsc

N = 10000
E = 320000
NC = 2
NS = 16
NW = NC * NS
CHUNK = 128
C = 80
EPW = C * CHUNK
CAP = NW * EPW
ACC_ROWS = 10240
ZPT = ACC_ROWS // NS
WB_FULL = N // CHUNK

_f32 = jnp.float32


def _zero_rows(buf, d):
    def row(i, _):
        for k in range(d // 16):
            buf[i, pl.ds(k * 16, 16)] = jnp.zeros((16,), _f32)
        return 0
    lax.fori_loop(0, CHUNK, row, 0)


def _ones_rows(buf, d):
    def row(i, _):
        for k in range(d // 16):
            buf[i, pl.ds(k * 16, 16)] = jnp.ones((16,), _f32)
        return 0
    lax.fori_loop(0, CHUNK, row, 0)


def _make_edge_scatter(d, do_gather):
    mesh = plsc.VectorSubcoreMesh(core_axis_name="c", subcore_axis_name="s")
    NB = 8
    scratch = [
        pltpu.VMEM((C, CHUNK), jnp.int32),
        pltpu.VMEM((C, CHUNK), jnp.int32),
        [pltpu.VMEM((CHUNK, d), _f32) for _ in range(NB)],
        pltpu.VMEM_SHARED((ACC_ROWS, d), _f32),
        pltpu.VMEM_SHARED((N, d), _f32) if do_gather else None,
        [pltpu.SemaphoreType.DMA for _ in range(NB)],
        [pltpu.SemaphoreType.DMA for _ in range(NB)],
    ]
    scratch = [s for s in scratch if s is not None]

    def body(u_hbm, src_hbm, dst_hbm, out_hbm, src_v, dst_v, bufs, acc,
             *rest):
        if do_gather:
            ush, gsem, ssem = rest
        else:
            gsem, ssem = rest
        cc = lax.axis_index("c")
        ss = lax.axis_index("s")
        w = cc * NS + ss

        if do_gather:
            pltpu.sync_copy(src_hbm.at[w], src_v)
            for k in range(5):
                g = ss * 5 + k
                r0 = g * CHUNK

                @pl.when(g < WB_FULL)
                def _():
                    pltpu.sync_copy(u_hbm.at[pl.ds(r0, CHUNK)],
                                    ush.at[pl.ds(r0, CHUNK)])

                @pl.when(g == WB_FULL)
                def _():
                    pltpu.sync_copy(u_hbm.at[pl.ds(r0, 16)],
                                    ush.at[pl.ds(r0, 16)])
        pltpu.sync_copy(dst_hbm.at[w], dst_v)

        _zero_rows(bufs[NB - 1], d)
        zbase = ss * ZPT
        for k in range(ZPT // CHUNK):
            pltpu.sync_copy(bufs[NB - 1], acc.at[pl.ds(zbase + k * CHUNK,
                                                       CHUNK)])
        if not do_gather:
            _ones_rows(bufs[0], d)
        plsc.subcore_barrier()

        if do_gather:
            m0 = bufs[0]
            for pj in range(NB // 2):
                pltpu.async_copy(ush.at[src_v.at[pj]], bufs[pj], gsem[pj])

            def step(i, _):
                for b in range(NB):
                    j = NB * i + b
                    kn = (b + NB // 2) % NB

                    @pl.when(j >= NB // 2)
                    def _():
                        pltpu.make_async_copy(
                            bufs[kn], acc.at[dst_v.at[j - NB // 2]],
                            ssem[kn]).wait()

                    @pl.when(j + NB // 2 < C)
                    def _():
                        pltpu.async_copy(ush.at[src_v.at[j + NB // 2]],
                                         bufs[kn], gsem[kn])

                    pltpu.make_async_copy(ush.at[src_v.at[j]], bufs[b],
                                          gsem[b]).wait()
                    pltpu.async_copy(bufs[b], acc.at[dst_v.at[j]], ssem[b],
                                     add=True)
                return 0
            lax.fori_loop(0, C // NB, step, 0)
            for j in range(C - NB // 2, C):
                pltpu.make_async_copy(bufs[j % NB], acc.at[dst_v.at[j]],
                                      ssem[j % NB]).wait()
        else:
            m0 = bufs[0]
            def step(i, _):
                for b in range(8):
                    pltpu.async_copy(m0, acc.at[dst_v.at[8 * i + b]],
                                     ssem[b % NB], add=True)
                for b in range(8):
                    pltpu.make_async_copy(m0, acc.at[dst_v.at[8 * i + b]],
                                          ssem[b % NB]).wait()
                return 0
            lax.fori_loop(0, C // 8, step, 0)

        plsc.subcore_barrier()

        for k in range(5):
            g = ss * 5 + k
            r0 = g * CHUNK

            @pl.when(g < WB_FULL)
            def _():
                pltpu.sync_copy(acc.at[pl.ds(r0, CHUNK)], m0)
                pltpu.sync_copy(m0, out_hbm.at[cc, pl.ds(r0, CHUNK)])

            @pl.when(g == WB_FULL)
            def _():
                pltpu.sync_copy(acc.at[pl.ds(r0, 16)], m0.at[pl.ds(0, 16)])
                pltpu.sync_copy(m0.at[pl.ds(0, 16)],
                                out_hbm.at[cc, pl.ds(r0, 16)])

    return pl.kernel(
        body,
        out_type=jax.ShapeDtypeStruct((NC, N, d), _f32),
        mesh=mesh,
        scratch_types=scratch,
        compiler_params=pltpu.CompilerParams(use_tc_tiling_on_sc=False),
    )


C2 = 160


def _make_edge_scatter_split():
    d = 32
    mesh = plsc.VectorSubcoreMesh(core_axis_name="c", subcore_axis_name="s")
    NB = 8
    scratch = [
        pltpu.VMEM((C2, CHUNK), jnp.int32),
        pltpu.VMEM((C2, CHUNK), jnp.int32),
        [pltpu.VMEM((CHUNK, d), _f32) for _ in range(NB)],
        pltpu.VMEM_SHARED((ACC_ROWS, d), _f32),
        pltpu.VMEM_SHARED((N, d), _f32),
        [pltpu.SemaphoreType.DMA for _ in range(NB)],
        [pltpu.SemaphoreType.DMA for _ in range(NB)],
    ]

    def body(ua_hbm, ub_hbm, src_hbm, dst_hbm, out_hbm, src_v, dst_v, bufs,
             acc, ush, gsem, ssem):
        cc = lax.axis_index("c")
        ss = lax.axis_index("s")

        pltpu.sync_copy(src_hbm.at[ss], src_v)
        for k in range(5):
            g = ss * 5 + k
            r0 = g * CHUNK

            @pl.when(g < WB_FULL)
            def _():
                @pl.when(cc == 0)
                def _():
                    pltpu.sync_copy(ua_hbm.at[pl.ds(r0, CHUNK)],
                                    ush.at[pl.ds(r0, CHUNK)])

                @pl.when(cc == 1)
                def _():
                    pltpu.sync_copy(ub_hbm.at[pl.ds(r0, CHUNK)],
                                    ush.at[pl.ds(r0, CHUNK)])

            @pl.when(g == WB_FULL)
            def _():
                @pl.when(cc == 0)
                def _():
                    pltpu.sync_copy(ua_hbm.at[pl.ds(r0, 16)],
                                    ush.at[pl.ds(r0, 16)])

                @pl.when(cc == 1)
                def _():
                    pltpu.sync_copy(ub_hbm.at[pl.ds(r0, 16)],
                                    ush.at[pl.ds(r0, 16)])
        pltpu.sync_copy(dst_hbm.at[ss], dst_v)

        _zero_rows(bufs[NB - 1], d)
        zbase = ss * ZPT
        for k in range(ZPT // CHUNK):
            pltpu.sync_copy(bufs[NB - 1], acc.at[pl.ds(zbase + k * CHUNK,
                                                       CHUNK)])
        plsc.subcore_barrier()

        m0 = bufs[0]
        for pj in range(NB // 2):
            pltpu.async_copy(ush.at[src_v.at[pj]], bufs[pj], gsem[pj])

        def step(i, _):
            for b in range(NB):
                j = NB * i + b
                kn = (b + NB // 2) % NB

                @pl.when(j >= NB // 2)
                def _():
                    pltpu.make_async_copy(
                        bufs[kn], acc.at[dst_v.at[j - NB // 2]],
                        ssem[kn]).wait()

                @pl.when(j + NB // 2 < C2)
                def _():
                    pltpu.async_copy(ush.at[src_v.at[j + NB // 2]], bufs[kn],
                                     gsem[kn])

                pltpu.make_async_copy(ush.at[src_v.at[j]], bufs[b],
                                      gsem[b]).wait()
                pltpu.async_copy(bufs[b], acc.at[dst_v.at[j]], ssem[b],
                                 add=True)
            return 0
        lax.fori_loop(0, C2 // NB, step, 0)
        for j in range(C2 - NB // 2, C2):
            pltpu.make_async_copy(bufs[j % NB], acc.at[dst_v.at[j]],
                                  ssem[j % NB]).wait()

        plsc.subcore_barrier()

        for k in range(5):
            g = ss * 5 + k
            r0 = g * CHUNK

            @pl.when(g < WB_FULL)
            def _():
                pltpu.sync_copy(acc.at[pl.ds(r0, CHUNK)], m0)
                pltpu.sync_copy(m0, out_hbm.at[cc, pl.ds(r0, CHUNK)])

            @pl.when(g == WB_FULL)
            def _():
                pltpu.sync_copy(acc.at[pl.ds(r0, 16)], m0.at[pl.ds(0, 16)])
                pltpu.sync_copy(m0.at[pl.ds(0, 16)],
                                out_hbm.at[cc, pl.ds(r0, 16)])

    return pl.kernel(
        body,
        out_type=jax.ShapeDtypeStruct((NC, N, 32), _f32),
        mesh=mesh,
        scratch_types=scratch,
        compiler_params=pltpu.CompilerParams(use_tc_tiling_on_sc=False),
    )


_R = 2000


def _dot(a, b):
    return jnp.dot(a, b, preferred_element_type=_f32,
                   precision=lax.Precision.HIGHEST)


def _tc_call(body, ins, out_widths, rows=N, r=_R):
    in_specs = []
    for a, kind in ins:
        if kind == "r":
            in_specs.append(pl.BlockSpec((r, a.shape[1]), lambda i: (i, 0)))
        elif kind == "p":
            in_specs.append(
                pl.BlockSpec((2, r, a.shape[2]), lambda i: (0, i, 0)))
        else:
            in_specs.append(
                pl.BlockSpec(a.shape, lambda i, nd=a.ndim: (0,) * nd))
    out_shape = [jax.ShapeDtypeStruct((rows, wd), _f32) for wd in out_widths]
    out_specs = [pl.BlockSpec((r, wd), lambda i: (i, 0)) for wd in out_widths]
    res = pl.pallas_call(
        body,
        grid=(rows // r,),
        in_specs=in_specs,
        out_specs=out_specs,
        out_shape=out_shape,
    )(*[a for a, _ in ins])
    return res


def _combine(p_ref, dinv, d2, hself):
    p = p_ref[...]
    return dinv * (p[0] + p[1]) + d2 * hself


def _stage0_body(pdeg, x, w1, dinv_o, d2_o, g1_o, u1_o):
    p = pdeg[...]
    deg = p[0, :, 0:1] + p[1, :, 0:1] + 1.0
    dinv = lax.rsqrt(deg)
    dinv_o[...] = dinv
    d2_o[...] = 1.0 / deg
    g1 = _dot(x[...], w1[...])
    g1_o[...] = g1
    u1_o[...] = dinv * g1


def _stageB_body(p, g1, dinv, d2, b1, h_o, u_o):
    h = _combine(p, dinv[...], d2[...], g1[...]) + b1[...]
    h_o[...] = h
    u_o[...] = dinv[...] * h


def _stageCD_body(p, hprev, dinv, d2, w, b, h_o, u_o):
    s = _combine(p, dinv[...], d2[...], hprev[...])
    h = _dot(s, w[...]) + b[...]
    h_o[...] = h
    u_o[...] = dinv[...] * h


def _stageD_body(p, hprev, dinv, d2, w, b, h_o, ua_o, ub_o):
    s = _combine(p, dinv[...], d2[...], hprev[...])
    h = _dot(s, w[...]) + b[...]
    h_o[...] = h
    u = dinv[...] * h
    ua_o[...] = u[:, :32]
    ub_o[...] = u[:, 32:]


def _combine2(p, dinv, d2, hself):
    v = p[...]
    return dinv * jnp.concatenate([v[0], v[1]], axis=1) + d2 * hself


def _stageE_body(p, h3, dinv, d2, w4, b4, w5, g5_o, ua_o, ub_o):
    s = _combine2(p, dinv[...], d2[...], h3[...])
    h4 = jnp.maximum(_dot(s, w4[...]) + b4[...], 0.0)
    g5 = _dot(h4, w5[...])
    g5_o[...] = g5
    u5 = dinv[...] * g5
    ua_o[...] = u5[:, :32]
    ub_o[...] = u5[:, 32:]


def _stageF_body(p, gprev, dinv, d2, bprev, wnext, g_o, u_o):
    h = _combine2(p, dinv[...], d2[...], gprev[...]) + bprev[...]
    g = _dot(h, wnext[...])
    g_o[...] = g
    u_o[...] = dinv[...] * g


def _stageFG_body(p, gprev, dinv, d2, bprev, wnext, g_o, u_o):
    h = _combine(p, dinv[...], d2[...], gprev[...]) + bprev[...]
    g = _dot(h, wnext[...])
    g_o[...] = g
    u_o[...] = dinv[...] * g


def _stageH_body(p, g7, dinv, d2, b7, h_o):
    h_o[...] = _combine(p, dinv[...], d2[...], g7[...]) + b7[...]


def _head_body(z, wl1, bl1, wl2, bl2, y_o):
    t = jnp.maximum(_dot(z[...], wl1[...]) + bl1[...], 0.0)
    y_o[...] = _dot(t, wl2[...]) + bl2[...]


@functools.partial(jax.jit, static_argnums=())
def _run(x, edge_index, W1, b1, W2, b2, W3, b3, W4, b4, W5, b5, W6, b6,
         W7, b7, Wl1, bl1, Wl2, bl2):
    pad = CAP - E
    src = jnp.concatenate(
        [edge_index[0], jnp.zeros((pad,), jnp.int32)]).reshape(NW, C, CHUNK)
    dst = jnp.concatenate(
        [edge_index[1],
         jnp.full((pad,), N, jnp.int32)]).reshape(NW, C, CHUNK)

    scat16 = _make_edge_scatter(16, True)
    scat32 = _make_edge_scatter(32, True)
    scat64s = _make_edge_scatter_split()
    degk = _make_edge_scatter(16, False)
    src16 = src.reshape(NS, C2, CHUNK)
    dst16 = dst.reshape(NS, C2, CHUNK)

    b1r, b2r, b3r, b4r, b5r, b6r, b7r = (
        b.reshape(1, -1) for b in (b1, b2, b3, b4, b5, b6, b7))
    bl1r, bl2r = bl1.reshape(1, -1), bl2.reshape(1, -1)

    dummy16 = jnp.zeros((N, 16), _f32)
    pdeg = degk(dummy16, src, dst)

    dinv, d2, g1, u1 = _tc_call(
        _stage0_body, [(pdeg, "p"), (x, "r"), (W1, "w")], [1, 1, 16, 16])

    p1 = scat16(u1, src, dst)
    h1, u2 = _tc_call(
        _stageB_body,
        [(p1, "p"), (g1, "r"), (dinv, "r"), (d2, "r"), (b1r, "w")], [16, 16])

    p2 = scat16(u2, src, dst)
    h2, u3 = _tc_call(
        _stageCD_body,
        [(p2, "p"), (h1, "r"), (dinv, "r"), (d2, "r"), (W2, "w"), (b2r, "w")],
        [32, 32])

    p3 = scat32(u3, src, dst)
    h3, u4a, u4b = _tc_call(
        _stageD_body,
        [(p3, "p"), (h2, "r"), (dinv, "r"), (d2, "r"), (W3, "w"), (b3r, "w")],
        [64, 32, 32])

    p4 = scat64s(u4a, u4b, src16, dst16)
    g5, u5a, u5b = _tc_call(
        _stageE_body,
        [(p4, "p"), (h3, "r"), (dinv, "r"), (d2, "r"),
         (W4, "w"), (b4r, "w"), (W5, "w")], [64, 32, 32])

    p5 = scat64s(u5a, u5b, src16, dst16)
    g6, u6 = _tc_call(
        _stageF_body,
        [(p5, "p"), (g5, "r"), (dinv, "r"), (d2, "r"),
         (b5r, "w"), (W6, "w")], [32, 32])

    p6 = scat32(u6, src, dst)
    g7, u7 = _tc_call(
        _stageFG_body,
        [(p6, "p"), (g6, "r"), (dinv, "r"), (d2, "r"), (b6r, "w"),
         (W7, "w")], [16, 16])

    p7 = scat16(u7, src, dst)
    (h7,) = _tc_call(
        _stageH_body,
        [(p7, "p"), (g7, "r"), (dinv, "r"), (d2, "r"), (b7r, "w")], [16])

    z = h7.reshape(N // 4, 64)
    (y,) = _tc_call(
        _head_body,
        [(z, "r"), (Wl1, "w"), (bl1r, "w"), (Wl2, "w"), (bl2r, "w")],
        [729], rows=N // 4, r=N // 4)
    return y.reshape(N // 4, 81, 9)


def kernel(x, edge_index, W1, b1, W2, b2, W3, b3, W4, b4, W5, b5, W6, b6,
           W7, b7, Wl1, bl1, Wl2, bl2):
    return _run(x, edge_index, W1, b1, W2, b2, W3, b3, W4, b4, W5, b5,
                W6, b6, W7, b7, Wl1, bl1, Wl2, bl2)


# speedup vs baseline: 1.0328x; 1.0052x over previous
import functools

import jax
import jax.numpy as jnp
from jax import lax
from jax.experimental import pallas as pl
from jax.experimental.pallas import tpu as pltpu
from jax.experimental.pallas import tpu_sc as plsc

N = 10000
E = 320000
NC = 2
NS = 16
NW = NC * NS
CHUNK = 128
C = 80
EPW = C * CHUNK
CAP = NW * EPW
ACC_ROWS = 10240
ZPT = ACC_ROWS // NS
WB_FULL = N // CHUNK

_f32 = jnp.float32


def _zero_rows(buf, d):
    def row(i, _):
        for k in range(d // 16):
            buf[i, pl.ds(k * 16, 16)] = jnp.zeros((16,), _f32)
        return 0
    lax.fori_loop(0, CHUNK, row, 0)


def _ones_rows(buf, d):
    def row(i, _):
        for k in range(d // 16):
            buf[i, pl.ds(k * 16, 16)] = jnp.ones((16,), _f32)
        return 0
    lax.fori_loop(0, CHUNK, row, 0)


def _make_edge_scatter(d, do_gather):
    mesh = plsc.VectorSubcoreMesh(core_axis_name="c", subcore_axis_name="s")
    NB = 4
    scratch = [
        pltpu.VMEM((C, CHUNK), jnp.int32),
        pltpu.VMEM((C, CHUNK), jnp.int32),
        [pltpu.VMEM((CHUNK, d), _f32) for _ in range(NB)],
        pltpu.VMEM_SHARED((ACC_ROWS, d), _f32),
        pltpu.VMEM_SHARED((N, d), _f32) if do_gather else None,
        [pltpu.SemaphoreType.DMA for _ in range(NB)],
        [pltpu.SemaphoreType.DMA for _ in range(NB)],
    ]
    scratch = [s for s in scratch if s is not None]

    def body(u_hbm, src_hbm, dst_hbm, out_hbm, src_v, dst_v, bufs, acc,
             *rest):
        if do_gather:
            ush, gsem, ssem = rest
        else:
            gsem, ssem = rest
        cc = lax.axis_index("c")
        ss = lax.axis_index("s")
        w = cc * NS + ss

        if do_gather:
            pltpu.sync_copy(src_hbm.at[w], src_v)
            for k in range(5):
                g = ss * 5 + k
                r0 = g * CHUNK

                @pl.when(g < WB_FULL)
                def _():
                    pltpu.sync_copy(u_hbm.at[pl.ds(r0, CHUNK)],
                                    ush.at[pl.ds(r0, CHUNK)])

                @pl.when(g == WB_FULL)
                def _():
                    pltpu.sync_copy(u_hbm.at[pl.ds(r0, 16)],
                                    ush.at[pl.ds(r0, 16)])
        pltpu.sync_copy(dst_hbm.at[w], dst_v)

        _zero_rows(bufs[NB - 1], d)
        zbase = ss * ZPT
        for k in range(ZPT // CHUNK):
            pltpu.sync_copy(bufs[NB - 1], acc.at[pl.ds(zbase + k * CHUNK,
                                                       CHUNK)])
        if not do_gather:
            _ones_rows(bufs[0], d)
        plsc.subcore_barrier()

        if do_gather:
            m0 = bufs[0]
            for pj in range(NB // 2):
                pltpu.async_copy(ush.at[src_v.at[pj]], bufs[pj], gsem[pj])

            def step(i, _):
                for b in range(NB):
                    j = NB * i + b
                    kn = (b + NB // 2) % NB

                    @pl.when(j >= NB // 2)
                    def _():
                        pltpu.make_async_copy(
                            bufs[kn], acc.at[dst_v.at[j - NB // 2]],
                            ssem[kn]).wait()

                    @pl.when(j + NB // 2 < C)
                    def _():
                        pltpu.async_copy(ush.at[src_v.at[j + NB // 2]],
                                         bufs[kn], gsem[kn])

                    pltpu.make_async_copy(ush.at[src_v.at[j]], bufs[b],
                                          gsem[b]).wait()
                    pltpu.async_copy(bufs[b], acc.at[dst_v.at[j]], ssem[b],
                                     add=True)
                return 0
            lax.fori_loop(0, C // NB, step, 0)
            for j in range(C - NB // 2, C):
                pltpu.make_async_copy(bufs[j % NB], acc.at[dst_v.at[j]],
                                      ssem[j % NB]).wait()
        else:
            m0 = bufs[0]
            def step(i, _):
                for b in range(8):
                    pltpu.async_copy(m0, acc.at[dst_v.at[8 * i + b]],
                                     ssem[b % NB], add=True)
                for b in range(8):
                    pltpu.make_async_copy(m0, acc.at[dst_v.at[8 * i + b]],
                                          ssem[b % NB]).wait()
                return 0
            lax.fori_loop(0, C // 8, step, 0)

        plsc.subcore_barrier()

        for k in range(5):
            g = ss * 5 + k
            r0 = g * CHUNK

            @pl.when(g < WB_FULL)
            def _():
                pltpu.sync_copy(acc.at[pl.ds(r0, CHUNK)], m0)
                pltpu.sync_copy(m0, out_hbm.at[cc, pl.ds(r0, CHUNK)])

            @pl.when(g == WB_FULL)
            def _():
                pltpu.sync_copy(acc.at[pl.ds(r0, 16)], m0.at[pl.ds(0, 16)])
                pltpu.sync_copy(m0.at[pl.ds(0, 16)],
                                out_hbm.at[cc, pl.ds(r0, 16)])

    return pl.kernel(
        body,
        out_type=jax.ShapeDtypeStruct((NC, N, d), _f32),
        mesh=mesh,
        scratch_types=scratch,
        compiler_params=pltpu.CompilerParams(use_tc_tiling_on_sc=False),
    )


C2 = 160


def _make_edge_scatter_split():
    d = 32
    mesh = plsc.VectorSubcoreMesh(core_axis_name="c", subcore_axis_name="s")
    NB = 4
    scratch = [
        pltpu.VMEM((C2, CHUNK), jnp.int32),
        pltpu.VMEM((C2, CHUNK), jnp.int32),
        [pltpu.VMEM((CHUNK, d), _f32) for _ in range(NB)],
        pltpu.VMEM_SHARED((ACC_ROWS, d), _f32),
        pltpu.VMEM_SHARED((N, d), _f32),
        [pltpu.SemaphoreType.DMA for _ in range(NB)],
        [pltpu.SemaphoreType.DMA for _ in range(NB)],
    ]

    def body(ua_hbm, ub_hbm, src_hbm, dst_hbm, out_hbm, src_v, dst_v, bufs,
             acc, ush, gsem, ssem):
        cc = lax.axis_index("c")
        ss = lax.axis_index("s")

        pltpu.sync_copy(src_hbm.at[ss], src_v)
        for k in range(5):
            g = ss * 5 + k
            r0 = g * CHUNK

            @pl.when(g < WB_FULL)
            def _():
                @pl.when(cc == 0)
                def _():
                    pltpu.sync_copy(ua_hbm.at[pl.ds(r0, CHUNK)],
                                    ush.at[pl.ds(r0, CHUNK)])

                @pl.when(cc == 1)
                def _():
                    pltpu.sync_copy(ub_hbm.at[pl.ds(r0, CHUNK)],
                                    ush.at[pl.ds(r0, CHUNK)])

            @pl.when(g == WB_FULL)
            def _():
                @pl.when(cc == 0)
                def _():
                    pltpu.sync_copy(ua_hbm.at[pl.ds(r0, 16)],
                                    ush.at[pl.ds(r0, 16)])

                @pl.when(cc == 1)
                def _():
                    pltpu.sync_copy(ub_hbm.at[pl.ds(r0, 16)],
                                    ush.at[pl.ds(r0, 16)])
        pltpu.sync_copy(dst_hbm.at[ss], dst_v)

        _zero_rows(bufs[NB - 1], d)
        zbase = ss * ZPT
        for k in range(ZPT // CHUNK):
            pltpu.sync_copy(bufs[NB - 1], acc.at[pl.ds(zbase + k * CHUNK,
                                                       CHUNK)])
        plsc.subcore_barrier()

        m0 = bufs[0]
        for pj in range(NB // 2):
            pltpu.async_copy(ush.at[src_v.at[pj]], bufs[pj], gsem[pj])

        def step(i, _):
            for b in range(NB):
                j = NB * i + b
                kn = (b + NB // 2) % NB

                @pl.when(j >= NB // 2)
                def _():
                    pltpu.make_async_copy(
                        bufs[kn], acc.at[dst_v.at[j - NB // 2]],
                        ssem[kn]).wait()

                @pl.when(j + NB // 2 < C2)
                def _():
                    pltpu.async_copy(ush.at[src_v.at[j + NB // 2]], bufs[kn],
                                     gsem[kn])

                pltpu.make_async_copy(ush.at[src_v.at[j]], bufs[b],
                                      gsem[b]).wait()
                pltpu.async_copy(bufs[b], acc.at[dst_v.at[j]], ssem[b],
                                 add=True)
            return 0
        lax.fori_loop(0, C2 // NB, step, 0)
        for j in range(C2 - NB // 2, C2):
            pltpu.make_async_copy(bufs[j % NB], acc.at[dst_v.at[j]],
                                  ssem[j % NB]).wait()

        plsc.subcore_barrier()

        for k in range(5):
            g = ss * 5 + k
            r0 = g * CHUNK

            @pl.when(g < WB_FULL)
            def _():
                pltpu.sync_copy(acc.at[pl.ds(r0, CHUNK)], m0)
                pltpu.sync_copy(m0, out_hbm.at[cc, pl.ds(r0, CHUNK)])

            @pl.when(g == WB_FULL)
            def _():
                pltpu.sync_copy(acc.at[pl.ds(r0, 16)], m0.at[pl.ds(0, 16)])
                pltpu.sync_copy(m0.at[pl.ds(0, 16)],
                                out_hbm.at[cc, pl.ds(r0, 16)])

    return pl.kernel(
        body,
        out_type=jax.ShapeDtypeStruct((NC, N, 32), _f32),
        mesh=mesh,
        scratch_types=scratch,
        compiler_params=pltpu.CompilerParams(use_tc_tiling_on_sc=False),
    )


_R = 2000


def _dot(a, b):
    return jnp.dot(a, b, preferred_element_type=_f32,
                   precision=lax.Precision.HIGHEST)


def _tc_call(body, ins, out_widths, rows=N, r=_R):
    in_specs = []
    for a, kind in ins:
        if kind == "r":
            in_specs.append(pl.BlockSpec((r, a.shape[1]), lambda i: (i, 0)))
        elif kind == "p":
            in_specs.append(
                pl.BlockSpec((2, r, a.shape[2]), lambda i: (0, i, 0)))
        else:
            in_specs.append(
                pl.BlockSpec(a.shape, lambda i, nd=a.ndim: (0,) * nd))
    out_shape = [jax.ShapeDtypeStruct((rows, wd), _f32) for wd in out_widths]
    out_specs = [pl.BlockSpec((r, wd), lambda i: (i, 0)) for wd in out_widths]
    res = pl.pallas_call(
        body,
        grid=(rows // r,),
        in_specs=in_specs,
        out_specs=out_specs,
        out_shape=out_shape,
    )(*[a for a, _ in ins])
    return res


def _combine(p_ref, dinv, d2, hself):
    p = p_ref[...]
    return dinv * (p[0] + p[1]) + d2 * hself


def _stage0_body(pdeg, x, w1, dinv_o, d2_o, g1_o, u1_o):
    p = pdeg[...]
    deg = p[0, :, 0:1] + p[1, :, 0:1] + 1.0
    dinv = lax.rsqrt(deg)
    dinv_o[...] = dinv
    d2_o[...] = 1.0 / deg
    g1 = _dot(x[...], w1[...])
    g1_o[...] = g1
    u1_o[...] = dinv * g1


def _stageB_body(p, g1, dinv, d2, b1, h_o, u_o):
    h = _combine(p, dinv[...], d2[...], g1[...]) + b1[...]
    h_o[...] = h
    u_o[...] = dinv[...] * h


def _stageCD_body(p, hprev, dinv, d2, w, b, h_o, u_o):
    s = _combine(p, dinv[...], d2[...], hprev[...])
    h = _dot(s, w[...]) + b[...]
    h_o[...] = h
    u_o[...] = dinv[...] * h


def _stageD_body(p, hprev, dinv, d2, w, b, h_o, ua_o, ub_o):
    s = _combine(p, dinv[...], d2[...], hprev[...])
    h = _dot(s, w[...]) + b[...]
    h_o[...] = h
    u = dinv[...] * h
    ua_o[...] = u[:, :32]
    ub_o[...] = u[:, 32:]


def _combine2(p, dinv, d2, hself):
    v = p[...]
    return dinv * jnp.concatenate([v[0], v[1]], axis=1) + d2 * hself


def _stageE_body(p, h3, dinv, d2, w4, b4, w5, g5_o, ua_o, ub_o):
    s = _combine2(p, dinv[...], d2[...], h3[...])
    h4 = jnp.maximum(_dot(s, w4[...]) + b4[...], 0.0)
    g5 = _dot(h4, w5[...])
    g5_o[...] = g5
    u5 = dinv[...] * g5
    ua_o[...] = u5[:, :32]
    ub_o[...] = u5[:, 32:]


def _stageF_body(p, gprev, dinv, d2, bprev, wnext, g_o, u_o):
    h = _combine2(p, dinv[...], d2[...], gprev[...]) + bprev[...]
    g = _dot(h, wnext[...])
    g_o[...] = g
    u_o[...] = dinv[...] * g


def _stageFG_body(p, gprev, dinv, d2, bprev, wnext, g_o, u_o):
    h = _combine(p, dinv[...], d2[...], gprev[...]) + bprev[...]
    g = _dot(h, wnext[...])
    g_o[...] = g
    u_o[...] = dinv[...] * g


def _stageH_body(p, g7, dinv, d2, b7, h_o):
    h_o[...] = _combine(p, dinv[...], d2[...], g7[...]) + b7[...]


def _head_body(z, wl1, bl1, wl2, bl2, y_o):
    t = jnp.maximum(_dot(z[...], wl1[...]) + bl1[...], 0.0)
    y_o[...] = _dot(t, wl2[...]) + bl2[...]


@functools.partial(jax.jit, static_argnums=())
def _run(x, edge_index, W1, b1, W2, b2, W3, b3, W4, b4, W5, b5, W6, b6,
         W7, b7, Wl1, bl1, Wl2, bl2):
    pad = CAP - E
    src = jnp.concatenate(
        [edge_index[0], jnp.zeros((pad,), jnp.int32)]).reshape(NW, C, CHUNK)
    dst = jnp.concatenate(
        [edge_index[1],
         jnp.full((pad,), N, jnp.int32)]).reshape(NW, C, CHUNK)

    scat16 = _make_edge_scatter(16, True)
    scat32 = _make_edge_scatter(32, True)
    scat64s = _make_edge_scatter_split()
    degk = _make_edge_scatter(16, False)
    src16 = src.reshape(NS, C2, CHUNK)
    dst16 = dst.reshape(NS, C2, CHUNK)

    b1r, b2r, b3r, b4r, b5r, b6r, b7r = (
        b.reshape(1, -1) for b in (b1, b2, b3, b4, b5, b6, b7))
    bl1r, bl2r = bl1.reshape(1, -1), bl2.reshape(1, -1)

    dummy16 = jnp.zeros((N, 16), _f32)
    pdeg = degk(dummy16, src, dst)

    dinv, d2, g1, u1 = _tc_call(
        _stage0_body, [(pdeg, "p"), (x, "r"), (W1, "w")], [1, 1, 16, 16])

    p1 = scat16(u1, src, dst)
    h1, u2 = _tc_call(
        _stageB_body,
        [(p1, "p"), (g1, "r"), (dinv, "r"), (d2, "r"), (b1r, "w")], [16, 16])

    p2 = scat16(u2, src, dst)
    h2, u3 = _tc_call(
        _stageCD_body,
        [(p2, "p"), (h1, "r"), (dinv, "r"), (d2, "r"), (W2, "w"), (b2r, "w")],
        [32, 32])

    p3 = scat32(u3, src, dst)
    h3, u4a, u4b = _tc_call(
        _stageD_body,
        [(p3, "p"), (h2, "r"), (dinv, "r"), (d2, "r"), (W3, "w"), (b3r, "w")],
        [64, 32, 32])

    p4 = scat64s(u4a, u4b, src16, dst16)
    g5, u5a, u5b = _tc_call(
        _stageE_body,
        [(p4, "p"), (h3, "r"), (dinv, "r"), (d2, "r"),
         (W4, "w"), (b4r, "w"), (W5, "w")], [64, 32, 32])

    p5 = scat64s(u5a, u5b, src16, dst16)
    g6, u6 = _tc_call(
        _stageF_body,
        [(p5, "p"), (g5, "r"), (dinv, "r"), (d2, "r"),
         (b5r, "w"), (W6, "w")], [32, 32])

    p6 = scat32(u6, src, dst)
    g7, u7 = _tc_call(
        _stageFG_body,
        [(p6, "p"), (g6, "r"), (dinv, "r"), (d2, "r"), (b6r, "w"),
         (W7, "w")], [16, 16])

    p7 = scat16(u7, src, dst)
    (h7,) = _tc_call(
        _stageH_body,
        [(p7, "p"), (g7, "r"), (dinv, "r"), (d2, "r"), (b7r, "w")], [16])

    z = h7.reshape(N // 4, 64)
    (y,) = _tc_call(
        _head_body,
        [(z, "r"), (Wl1, "w"), (bl1r, "w"), (Wl2, "w"), (bl2r, "w")],
        [729], rows=N // 4, r=N // 4)
    return y.reshape(N // 4, 81, 9)


def kernel(x, edge_index, W1, b1, W2, b2, W3, b3, W4, b4, W5, b5, W6, b6,
           W7, b7, Wl1, bl1, Wl2, bl2):
    return _run(x, edge_index, W1, b1, W2, b2, W3, b3, W4, b4, W5, b5,
                W6, b6, W7, b7, Wl1, bl1, Wl2, bl2)
